# Initial kernel scaffold; baseline (speedup 1.0000x reference)
import jax
import jax.numpy as jnp
from jax.experimental import pallas as pl


def kernel(x, centers, biases):
    raise NotImplementedError("write your pallas kernel here")


# speedup vs baseline: 10.0193x; 10.0193x over previous
import jax
import jax.numpy as jnp
from jax import lax
from jax.experimental import pallas as pl

_DIM = 256
_NUM_CODEBOOKS = 8
_CODEBOOK_SIZE = 1024


def _body(x_ref, c_ref, b_ref, err_ref, xss_ref):
    T = x_ref.shape[0]
    xb = x_ref[:]
    logits = lax.dot_general(
        xb, c_ref[:], (((1,), (1,)), ((), ())),
        preferred_element_type=jnp.float32,
    )
    logits = logits + b_ref[:]

    iota = lax.broadcasted_iota(jnp.int32, (T, _CODEBOOK_SIZE), 1)
    recon = jnp.zeros((T, _DIM), dtype=jnp.float32)
    for c in range(_NUM_CODEBOOKS):
        lg = logits[:, c * _CODEBOOK_SIZE:(c + 1) * _CODEBOOK_SIZE]
        m = jnp.max(lg, axis=1, keepdims=True)
        masked = jnp.where(lg == m, iota, _CODEBOOK_SIZE)
        idx = jnp.min(masked, axis=1, keepdims=True)
        oh = (iota == idx).astype(jnp.float32)
        recon = recon + jnp.dot(
            oh, c_ref[c * _CODEBOOK_SIZE:(c + 1) * _CODEBOOK_SIZE, :],
            preferred_element_type=jnp.float32,
        )

    err = recon - xb
    err_ss = jnp.sum(err * err).reshape(1, 1)
    x_ss = jnp.sum(xb * xb).reshape(1, 1)

    @pl.when(pl.program_id(0) == 0)
    def _init():
        err_ref[:, :] = err_ss
        xss_ref[:, :] = x_ss

    @pl.when(pl.program_id(0) != 0)
    def _acc():
        err_ref[:, :] += err_ss
        xss_ref[:, :] += x_ss


def kernel(x, centers, biases):
    xr = x.reshape(-1, _DIM)
    B = xr.shape[0]
    T = 512
    grid = B // T
    c2 = centers.reshape(_NUM_CODEBOOKS * _CODEBOOK_SIZE, _DIM)
    b2 = biases.reshape(1, _NUM_CODEBOOKS * _CODEBOOK_SIZE)

    err_ss, x_ss = pl.pallas_call(
        _body,
        grid=(grid,),
        in_specs=[
            pl.BlockSpec((T, _DIM), lambda i: (i, 0)),
            pl.BlockSpec(c2.shape, lambda i: (0, 0)),
            pl.BlockSpec(b2.shape, lambda i: (0, 0)),
        ],
        out_specs=[
            pl.BlockSpec((1, 1), lambda i: (0, 0)),
            pl.BlockSpec((1, 1), lambda i: (0, 0)),
        ],
        out_shape=[
            jax.ShapeDtypeStruct((1, 1), jnp.float32),
            jax.ShapeDtypeStruct((1, 1), jnp.float32),
        ],
    )(xr, c2, b2)
    return err_ss[0, 0] / (x_ss[0, 0] + 1e-20)


# speedup vs baseline: 12.8348x; 1.2810x over previous
import jax
import jax.numpy as jnp
from jax import lax
from jax.experimental import pallas as pl

_DIM = 256
_NUM_CODEBOOKS = 8
_CODEBOOK_SIZE = 1024


def _body(x_ref, c_ref, b_ref, err_ref, xss_ref):
    T = x_ref.shape[0]
    xb = x_ref[:]
    logits = lax.dot_general(
        xb, c_ref[:], (((1,), (1,)), ((), ())),
        preferred_element_type=jnp.float32,
    )
    logits = logits + b_ref[:]

    recon = jnp.zeros((T, _DIM), dtype=jnp.float32)
    for c in range(_NUM_CODEBOOKS):
        lg = logits[:, c * _CODEBOOK_SIZE:(c + 1) * _CODEBOOK_SIZE]
        m = jnp.max(lg, axis=1, keepdims=True)
        oh = (lg == m).astype(jnp.float32)
        recon = recon + jnp.dot(
            oh, c_ref[c * _CODEBOOK_SIZE:(c + 1) * _CODEBOOK_SIZE, :],
            preferred_element_type=jnp.float32,
        )

    err = recon - xb
    err_ss = jnp.sum(err * err).reshape(1, 1)
    x_ss = jnp.sum(xb * xb).reshape(1, 1)

    @pl.when(pl.program_id(0) == 0)
    def _init():
        err_ref[:, :] = err_ss
        xss_ref[:, :] = x_ss

    @pl.when(pl.program_id(0) != 0)
    def _acc():
        err_ref[:, :] += err_ss
        xss_ref[:, :] += x_ss


def kernel(x, centers, biases):
    xr = x.reshape(-1, _DIM)
    B = xr.shape[0]
    T = 512
    grid = B // T
    c2 = centers.reshape(_NUM_CODEBOOKS * _CODEBOOK_SIZE, _DIM)
    b2 = biases.reshape(1, _NUM_CODEBOOKS * _CODEBOOK_SIZE)

    err_ss, x_ss = pl.pallas_call(
        _body,
        grid=(grid,),
        in_specs=[
            pl.BlockSpec((T, _DIM), lambda i: (i, 0)),
            pl.BlockSpec(c2.shape, lambda i: (0, 0)),
            pl.BlockSpec(b2.shape, lambda i: (0, 0)),
        ],
        out_specs=[
            pl.BlockSpec((1, 1), lambda i: (0, 0)),
            pl.BlockSpec((1, 1), lambda i: (0, 0)),
        ],
        out_shape=[
            jax.ShapeDtypeStruct((1, 1), jnp.float32),
            jax.ShapeDtypeStruct((1, 1), jnp.float32),
        ],
    )(xr, c2, b2)
    return err_ss[0, 0] / (x_ss[0, 0] + 1e-20)
